# Initial kernel scaffold; baseline (speedup 1.0000x reference)
import functools

import jax
import jax.numpy as jnp
from jax import lax
from jax.experimental import pallas as pl
from jax.experimental.pallas import tpu as pltpu
from jax.experimental.pallas import tpu_sc as plsc

SEQ_LEN = 200
EMBED_DIM = 32
CHUNK = 128


def _sc_kernel_body(n_chunks, idx_hbm, pos2_hbm, table_hbm, out_hbm,
                    idx_v, pos2_v, buf, sem):
    info = plsc.get_sparse_core_info()
    nc = info.num_cores
    wid = lax.axis_index("s") * nc + lax.axis_index("c")

    pltpu.sync_copy(idx_hbm.at[pl.ds(wid * n_chunks, n_chunks)], idx_v)
    pltpu.sync_copy(pos2_hbm, pos2_v)

    base = wid * (n_chunks * CHUNK)

    @pl.loop(0, n_chunks)
    def _chunk(c):
        pltpu.async_copy(table_hbm.at[idx_v.at[c]], buf, sem).wait()

        phi = lax.rem(c * CHUNK, SEQ_LEN)

        @pl.loop(0, CHUNK, unroll=8)
        def _row(r):
            for h in range(EMBED_DIM // 16):
                sl = pl.ds(h * 16, 16)
                buf[r, sl] = buf[r, sl] + pos2_v[phi + r, sl]

        pltpu.sync_copy(buf, out_hbm.at[pl.ds(base + c * CHUNK, CHUNK)])


def kernel(inputs, table, pos):
    B, L = inputs.shape
    V, D = table.shape
    total = B * L
    n_workers = 32
    rows_per_w = total // n_workers
    n_chunks = rows_per_w // CHUNK

    idx = inputs.reshape(total // CHUNK, CHUNK).astype(jnp.int32)
    pos2 = jnp.concatenate([pos, pos], axis=0)

    mesh = plsc.VectorSubcoreMesh(core_axis_name="c", subcore_axis_name="s")
    k = pl.kernel(
        functools.partial(_sc_kernel_body, n_chunks),
        out_type=jax.ShapeDtypeStruct((total, D), jnp.float32),
        mesh=mesh,
        scratch_types=[
            pltpu.VMEM((n_chunks, CHUNK), jnp.int32),
            pltpu.VMEM((2 * L, D), jnp.float32),
            pltpu.VMEM((CHUNK, D), jnp.float32),
            pltpu.SemaphoreType.DMA,
        ],
    )
    out = k(idx, pos2, table)
    return out.reshape(B, L, D)


# speedup vs baseline: 1.0868x; 1.0868x over previous
import functools

import jax
import jax.numpy as jnp
from jax import lax
from jax.experimental import pallas as pl
from jax.experimental.pallas import tpu as pltpu
from jax.experimental.pallas import tpu_sc as plsc

SEQ_LEN = 200
EMBED_DIM = 32
CHUNK = 128


def _sc_kernel_body(n_chunks, nc, idx_hbm, pos2_hbm, table_hbm, out_hbm,
                    idx_v, pos2_v, buf, sem):
    wid = lax.axis_index("s") * nc + lax.axis_index("c")

    pltpu.sync_copy(idx_hbm.at[pl.ds(wid * n_chunks, n_chunks)], idx_v)
    pltpu.sync_copy(pos2_hbm, pos2_v)

    base = wid * (n_chunks * CHUNK)

    @pl.loop(0, n_chunks)
    def _chunk(c):
        pltpu.async_copy(table_hbm.at[idx_v.at[c]], buf, sem).wait()

        phi = lax.rem(c * CHUNK, SEQ_LEN)

        @pl.loop(0, CHUNK, unroll=8)
        def _row(r):
            for h in range(EMBED_DIM // 16):
                sl = pl.ds(h * 16, 16)
                buf[r, sl] = buf[r, sl] + pos2_v[phi + r, sl]

        pltpu.sync_copy(buf, out_hbm.at[pl.ds(base + c * CHUNK, CHUNK)])


def kernel(inputs, table, pos):
    B, L = inputs.shape
    V, D = table.shape
    total = B * L
    mesh = plsc.VectorSubcoreMesh(
        core_axis_name="c", subcore_axis_name="s",
        num_cores=2, num_subcores=16)
    n_workers = mesh.num_cores * mesh.num_subcores
    rows_per_w = total // n_workers
    n_chunks = rows_per_w // CHUNK

    idx = inputs.reshape(total // CHUNK, CHUNK).astype(jnp.int32)
    pos2 = jnp.concatenate([pos, pos], axis=0)

    k = pl.kernel(
        functools.partial(_sc_kernel_body, n_chunks, mesh.num_cores),
        out_type=jax.ShapeDtypeStruct((total, D), jnp.float32),
        mesh=mesh,
        scratch_types=[
            pltpu.VMEM((n_chunks, CHUNK), jnp.int32),
            pltpu.VMEM((2 * L, D), jnp.float32),
            pltpu.VMEM((CHUNK, D), jnp.float32),
            pltpu.SemaphoreType.DMA,
        ],
        compiler_params=pltpu.CompilerParams(use_tc_tiling_on_sc=False),
    )
    out = k(idx, pos2, table)
    return out.reshape(B, L, D)


# speedup vs baseline: 1.3139x; 1.2089x over previous
import functools

import jax
import jax.numpy as jnp
from jax import lax
from jax.experimental import pallas as pl
from jax.experimental.pallas import tpu as pltpu
from jax.experimental.pallas import tpu_sc as plsc

SEQ_LEN = 200
EMBED_DIM = 32
CHUNK = 128
NBUF = 4
AHEAD = 2


def _sc_kernel_body(n_chunks, nc, idx_hbm, pos2_hbm, table_hbm, out_hbm,
                    idx_v, pos2_v, buf, gsem, ssem):
    wid = lax.axis_index("s") * nc + lax.axis_index("c")

    pltpu.sync_copy(idx_hbm.at[pl.ds(wid * n_chunks, n_chunks)], idx_v)
    pltpu.sync_copy(pos2_hbm, pos2_v)

    base = wid * (n_chunks * CHUNK)

    def gather(c, slot):
        return pltpu.make_async_copy(
            table_hbm.at[idx_v.at[c]], buf.at[slot], gsem.at[slot])

    def store(c, slot):
        return pltpu.make_async_copy(
            buf.at[slot], out_hbm.at[pl.ds(base + c * CHUNK, CHUNK)],
            ssem.at[slot])

    for b in range(AHEAD):
        gather(b, b).start()

    @pl.loop(0, n_chunks // NBUF)
    def _grp(g):
        for b in range(NBUF):
            c = g * NBUF + b
            sf = (b + AHEAD) % NBUF

            @pl.when(c + AHEAD < n_chunks)
            def _():
                @pl.when(c + AHEAD >= NBUF)
                def _():
                    store(0, sf).wait()
                gather(c + AHEAD, sf).start()

            gather(c, b).wait()

            phi = lax.rem(c * CHUNK, SEQ_LEN)

            @pl.loop(0, CHUNK, unroll=8)
            def _row(r):
                for h in range(EMBED_DIM // 16):
                    sl = pl.ds(h * 16, 16)
                    plsc.addupdate(buf.at[b, r, sl], pos2_v[phi + r, sl])

            store(c, b).start()

    for b in range(NBUF):
        store(0, b).wait()


def kernel(inputs, table, pos):
    B, L = inputs.shape
    V, D = table.shape
    total = B * L
    mesh = plsc.VectorSubcoreMesh(
        core_axis_name="c", subcore_axis_name="s",
        num_cores=2, num_subcores=16)
    n_workers = mesh.num_cores * mesh.num_subcores
    rows_per_w = total // n_workers
    n_chunks = rows_per_w // CHUNK

    idx = inputs.reshape(total // CHUNK, CHUNK).astype(jnp.int32)
    pos2 = jnp.concatenate([pos, pos], axis=0)

    k = pl.kernel(
        functools.partial(_sc_kernel_body, n_chunks, mesh.num_cores),
        out_type=jax.ShapeDtypeStruct((total, D), jnp.float32),
        mesh=mesh,
        scratch_types=[
            pltpu.VMEM((n_chunks, CHUNK), jnp.int32),
            pltpu.VMEM((2 * L, D), jnp.float32),
            pltpu.VMEM((NBUF, CHUNK, D), jnp.float32),
            pltpu.SemaphoreType.DMA((NBUF,)),
            pltpu.SemaphoreType.DMA((NBUF,)),
        ],
        compiler_params=pltpu.CompilerParams(use_tc_tiling_on_sc=False),
    )
    out = k(idx, pos2, table)
    return out.reshape(B, L, D)


# speedup vs baseline: 1.4695x; 1.1184x over previous
import functools

import jax
import jax.numpy as jnp
from jax import lax
from jax.experimental import pallas as pl
from jax.experimental.pallas import tpu as pltpu
from jax.experimental.pallas import tpu_sc as plsc

SEQ_LEN = 200
EMBED_DIM = 32
CHUNK = 128
NBUF = 4
AHEAD = 2


def _sc_kernel_body(n_chunks, nc, idx_hbm, pos2_hbm, table_hbm, out_hbm,
                    idx_v, pos2_sh, buf, gsem, ssem):
    wid = lax.axis_index("s") * nc + lax.axis_index("c")

    pltpu.sync_copy(idx_hbm.at[pl.ds(wid * n_chunks, n_chunks)], idx_v)

    @pl.when(lax.axis_index("s") == 0)
    def _():
        pltpu.sync_copy(pos2_hbm, pos2_sh)

    plsc.subcore_barrier()

    base = wid * (n_chunks * CHUNK)

    def gather(c, slot):
        return pltpu.make_async_copy(
            table_hbm.at[idx_v.at[c]], buf.at[slot], gsem.at[slot])

    def store(c, slot):
        return pltpu.make_async_copy(
            buf.at[slot], out_hbm.at[pl.ds(base + c * CHUNK, CHUNK)],
            ssem.at[slot])

    def pos_init(c, slot):
        phi = lax.rem(c * CHUNK, SEQ_LEN)
        pltpu.sync_copy(pos2_sh.at[pl.ds(phi, CHUNK)], buf.at[slot])

    for b in range(AHEAD):
        pos_init(b, b)
        gather(b, b).start(add=True)

    @pl.loop(0, n_chunks // NBUF)
    def _grp(g):
        for b in range(NBUF):
            c = g * NBUF + b
            sf = (b + AHEAD) % NBUF

            @pl.when(c + AHEAD < n_chunks)
            def _():
                @pl.when(c + AHEAD >= NBUF)
                def _():
                    store(0, sf).wait()
                pos_init(c + AHEAD, sf)
                gather(c + AHEAD, sf).start(add=True)

            gather(c, b).wait()
            store(c, b).start()

    for b in range(NBUF):
        store(0, b).wait()


def kernel(inputs, table, pos):
    B, L = inputs.shape
    V, D = table.shape
    total = B * L
    mesh = plsc.VectorSubcoreMesh(
        core_axis_name="c", subcore_axis_name="s",
        num_cores=2, num_subcores=16)
    n_workers = mesh.num_cores * mesh.num_subcores
    rows_per_w = total // n_workers
    n_chunks = rows_per_w // CHUNK

    idx = inputs.reshape(total // CHUNK, CHUNK).astype(jnp.int32)
    pos2 = jnp.concatenate([pos, pos], axis=0)

    k = pl.kernel(
        functools.partial(_sc_kernel_body, n_chunks, mesh.num_cores),
        out_type=jax.ShapeDtypeStruct((total, D), jnp.float32),
        mesh=mesh,
        scratch_types=[
            pltpu.VMEM((n_chunks, CHUNK), jnp.int32),
            pltpu.VMEM_SHARED((2 * L, D), jnp.float32),
            pltpu.VMEM((NBUF, CHUNK, D), jnp.float32),
            pltpu.SemaphoreType.DMA((NBUF,)),
            pltpu.SemaphoreType.DMA((NBUF,)),
        ],
        compiler_params=pltpu.CompilerParams(use_tc_tiling_on_sc=False),
    )
    out = k(idx, pos2, table)
    return out.reshape(B, L, D)


# speedup vs baseline: 1.4983x; 1.0196x over previous
import functools

import jax
import jax.numpy as jnp
from jax import lax
from jax.experimental import pallas as pl
from jax.experimental.pallas import tpu as pltpu
from jax.experimental.pallas import tpu_sc as plsc

SEQ_LEN = 200
EMBED_DIM = 32
CHUNK = 128
NBUF = 8
AHEAD = 6


def _sc_kernel_body(n_chunks, nc, idx_hbm, pos2_hbm, table_hbm, out_hbm,
                    idx_v, pos2_sh, buf, gsem, ssem):
    wid = lax.axis_index("s") * nc + lax.axis_index("c")

    pltpu.sync_copy(idx_hbm.at[pl.ds(wid * n_chunks, n_chunks)], idx_v)

    @pl.when(lax.axis_index("s") == 0)
    def _():
        pltpu.sync_copy(pos2_hbm, pos2_sh)

    plsc.subcore_barrier()

    base = wid * (n_chunks * CHUNK)

    def gather(c, slot):
        return pltpu.make_async_copy(
            table_hbm.at[idx_v.at[c]], buf.at[slot], gsem.at[slot])

    def store(c, slot):
        return pltpu.make_async_copy(
            buf.at[slot], out_hbm.at[pl.ds(base + c * CHUNK, CHUNK)],
            ssem.at[slot])

    def pos_init(c, slot):
        phi = lax.rem(c * CHUNK, SEQ_LEN)
        pltpu.sync_copy(pos2_sh.at[pl.ds(phi, CHUNK)], buf.at[slot])

    for b in range(AHEAD):
        pos_init(b, b)
        gather(b, b).start(add=True)

    @pl.loop(0, n_chunks // NBUF)
    def _grp(g):
        for b in range(NBUF):
            c = g * NBUF + b
            sf = (b + AHEAD) % NBUF

            @pl.when(c + AHEAD < n_chunks)
            def _():
                @pl.when(c + AHEAD >= NBUF)
                def _():
                    store(0, sf).wait()
                pos_init(c + AHEAD, sf)
                gather(c + AHEAD, sf).start(add=True)

            gather(c, b).wait()
            store(c, b).start()

    for b in range(NBUF):
        store(0, b).wait()


def kernel(inputs, table, pos):
    B, L = inputs.shape
    V, D = table.shape
    total = B * L
    mesh = plsc.VectorSubcoreMesh(
        core_axis_name="c", subcore_axis_name="s",
        num_cores=2, num_subcores=16)
    n_workers = mesh.num_cores * mesh.num_subcores
    rows_per_w = total // n_workers
    n_chunks = rows_per_w // CHUNK

    idx = inputs.reshape(total // CHUNK, CHUNK).astype(jnp.int32)
    pos2 = jnp.concatenate([pos, pos], axis=0)

    k = pl.kernel(
        functools.partial(_sc_kernel_body, n_chunks, mesh.num_cores),
        out_type=jax.ShapeDtypeStruct((total, D), jnp.float32),
        mesh=mesh,
        scratch_types=[
            pltpu.VMEM((n_chunks, CHUNK), jnp.int32),
            pltpu.VMEM_SHARED((2 * L, D), jnp.float32),
            pltpu.VMEM((NBUF, CHUNK, D), jnp.float32),
            pltpu.SemaphoreType.DMA((NBUF,)),
            pltpu.SemaphoreType.DMA((NBUF,)),
        ],
        compiler_params=pltpu.CompilerParams(use_tc_tiling_on_sc=False),
    )
    out = k(idx, pos2, table)
    return out.reshape(B, L, D)


# speedup vs baseline: 1.4994x; 1.0008x over previous
import functools

import jax
import jax.numpy as jnp
from jax import lax
from jax.experimental import pallas as pl
from jax.experimental.pallas import tpu as pltpu
from jax.experimental.pallas import tpu_sc as plsc

SEQ_LEN = 200
EMBED_DIM = 32
CHUNK = 128
NBUF = 10
AHEAD = 8


def _sc_kernel_body(n_chunks, nc, idx_hbm, pos2_hbm, table_hbm, out_hbm,
                    idx_v, pos2_sh, buf, gsem, ssem):
    wid = lax.axis_index("s") * nc + lax.axis_index("c")

    pltpu.sync_copy(idx_hbm.at[pl.ds(wid * n_chunks, n_chunks)], idx_v)

    @pl.when(lax.axis_index("s") == 0)
    def _():
        pltpu.sync_copy(pos2_hbm, pos2_sh)

    plsc.subcore_barrier()

    base = wid * (n_chunks * CHUNK)

    def gather(c, slot):
        return pltpu.make_async_copy(
            table_hbm.at[idx_v.at[c]], buf.at[slot], gsem.at[slot])

    def store(c, slot):
        return pltpu.make_async_copy(
            buf.at[slot], out_hbm.at[pl.ds(base + c * CHUNK, CHUNK)],
            ssem.at[slot])

    def pos_init(c, slot):
        phi = lax.rem(c * CHUNK, SEQ_LEN)
        pltpu.sync_copy(pos2_sh.at[pl.ds(phi, CHUNK)], buf.at[slot])

    for b in range(AHEAD):
        pos_init(b, b)
        gather(b, b).start(add=True)

    @pl.loop(0, n_chunks // NBUF)
    def _grp(g):
        for b in range(NBUF):
            c = g * NBUF + b
            sf = (b + AHEAD) % NBUF

            @pl.when(c + AHEAD < n_chunks)
            def _():
                @pl.when(c + AHEAD >= NBUF)
                def _():
                    store(0, sf).wait()
                pos_init(c + AHEAD, sf)
                gather(c + AHEAD, sf).start(add=True)

            gather(c, b).wait()
            store(c, b).start()

    for b in range(NBUF):
        store(0, b).wait()


def kernel(inputs, table, pos):
    B, L = inputs.shape
    V, D = table.shape
    total = B * L
    mesh = plsc.VectorSubcoreMesh(
        core_axis_name="c", subcore_axis_name="s",
        num_cores=2, num_subcores=16)
    n_workers = mesh.num_cores * mesh.num_subcores
    rows_per_w = total // n_workers
    n_chunks = rows_per_w // CHUNK

    idx = inputs.reshape(total // CHUNK, CHUNK).astype(jnp.int32)
    pos2 = jnp.concatenate([pos, pos], axis=0)

    k = pl.kernel(
        functools.partial(_sc_kernel_body, n_chunks, mesh.num_cores),
        out_type=jax.ShapeDtypeStruct((total, D), jnp.float32),
        mesh=mesh,
        scratch_types=[
            pltpu.VMEM((n_chunks, CHUNK), jnp.int32),
            pltpu.VMEM_SHARED((2 * L, D), jnp.float32),
            pltpu.VMEM((NBUF, CHUNK, D), jnp.float32),
            pltpu.SemaphoreType.DMA((NBUF,)),
            pltpu.SemaphoreType.DMA((NBUF,)),
        ],
        compiler_params=pltpu.CompilerParams(use_tc_tiling_on_sc=False),
    )
    out = k(idx, pos2, table)
    return out.reshape(B, L, D)
